# baseline (speedup 1.0000x reference)
import jax
import jax.numpy as jnp
from jax.experimental import pallas as pl
from jax.experimental.pallas import tpu as pltpu

_BATCH = 16384
_N_ORB = 64
_D_IN = 128
_D_H = 2048
_TB = 2048
_NC = 8


def _mlp_tile(
    a_ref, be_ref, w1_ref, b1_ref, w2_ref, b2_ref, w3_ref, b3_ref, out_ref
):
    c = _TB // _NC
    w1a = w1_ref[:_N_ORB, :]
    w1b = w1_ref[_N_ORB:, :]
    w2 = w2_ref[...]
    b1v = b1_ref[...]
    b2v = b2_ref[...]
    w3v = w3_ref[...]
    b3v = b3_ref[0, 0]
    z1 = []
    for i in range(_NC):
        av = a_ref[i * c : (i + 1) * c, :].astype(jnp.bfloat16)
        bv = be_ref[i * c : (i + 1) * c, :].astype(jnp.bfloat16)
        z1.append(
            jax.lax.dot(av, w1a, preferred_element_type=jnp.float32)
            + jax.lax.dot(bv, w1b, preferred_element_type=jnp.float32)
        )
    for i in range(_NC):
        h1 = jnp.tanh(z1[i] + b1v).astype(jnp.bfloat16)
        z2 = jax.lax.dot(h1, w2, preferred_element_type=jnp.float32)
        h2 = jnp.tanh(z2 + b2v)
        y = jnp.sum(h2 * w3v, axis=1) + b3v
        out_ref[i * c : (i + 1) * c] = 2.0 * y


def kernel(alpha, beta, W1, b1, W2, b2, W3, b3):
    w1bf = W1.astype(jnp.bfloat16)
    w2bf = W2.astype(jnp.bfloat16)
    b1r = b1.reshape(1, _D_H)
    b2r = b2.reshape(1, _D_H)
    w3r = W3.reshape(1, _D_H)
    b3r = b3.reshape(1, 1)

    grid = (_BATCH // _TB,)
    out = pl.pallas_call(
        _mlp_tile,
        grid=grid,
        in_specs=[
            pl.BlockSpec((_TB, _N_ORB), lambda i: (i, 0)),
            pl.BlockSpec((_TB, _N_ORB), lambda i: (i, 0)),
            pl.BlockSpec((_D_IN, _D_H), lambda i: (0, 0)),
            pl.BlockSpec((1, _D_H), lambda i: (0, 0)),
            pl.BlockSpec((_D_H, _D_H), lambda i: (0, 0)),
            pl.BlockSpec((1, _D_H), lambda i: (0, 0)),
            pl.BlockSpec((1, _D_H), lambda i: (0, 0)),
            pl.BlockSpec((1, 1), lambda i: (0, 0)),
        ],
        out_specs=pl.BlockSpec((_TB,), lambda i: (i,)),
        out_shape=jax.ShapeDtypeStruct((_BATCH,), jnp.float32),
        compiler_params=pltpu.CompilerParams(
            dimension_semantics=("arbitrary",),
        ),
    )(alpha, beta, w1bf, b1r, w2bf, b2r, w3r, b3r)
    return out


# speedup vs baseline: 1.0439x; 1.0439x over previous
import jax
import jax.numpy as jnp
from jax.experimental import pallas as pl
from jax.experimental.pallas import tpu as pltpu

_BATCH = 16384
_N_ORB = 64
_D_IN = 128
_D_H = 2048
_TB = 2048
_NC = 8


def _mlp_tile(
    a_ref, be_ref, w1_ref, b1_ref, w2_ref, b2_ref, w3_ref, b3_ref, out_ref
):
    c = _TB // _NC
    w1a = w1_ref[:_N_ORB, :]
    w1b = w1_ref[_N_ORB:, :]
    w2 = w2_ref[...]
    b1v = b1_ref[...]
    b2v = b2_ref[...]
    w3v = w3_ref[...]
    b3v = b3_ref[0, 0]
    z1 = []
    for i in range(_NC):
        av = a_ref[i * c : (i + 1) * c, :].astype(jnp.float32)
        bv = be_ref[i * c : (i + 1) * c, :].astype(jnp.float32)
        z1.append(
            jax.lax.dot(av, w1a, preferred_element_type=jnp.float32)
            + jax.lax.dot(bv, w1b, preferred_element_type=jnp.float32)
        )
    for i in range(_NC):
        h1 = jnp.tanh(z1[i] + b1v)
        z2 = jax.lax.dot(h1, w2, preferred_element_type=jnp.float32)
        h2 = jnp.tanh(z2 + b2v)
        y = jnp.sum(h2 * w3v, axis=1) + b3v
        out_ref[i * c : (i + 1) * c] = 2.0 * y


def kernel(alpha, beta, W1, b1, W2, b2, W3, b3):
    b1r = b1.reshape(1, _D_H)
    b2r = b2.reshape(1, _D_H)
    w3r = W3.reshape(1, _D_H)
    b3r = b3.reshape(1, 1)

    grid = (_BATCH // _TB,)
    out = pl.pallas_call(
        _mlp_tile,
        grid=grid,
        in_specs=[
            pl.BlockSpec((_TB, _N_ORB), lambda i: (i, 0)),
            pl.BlockSpec((_TB, _N_ORB), lambda i: (i, 0)),
            pl.BlockSpec((_D_IN, _D_H), lambda i: (0, 0)),
            pl.BlockSpec((1, _D_H), lambda i: (0, 0)),
            pl.BlockSpec((_D_H, _D_H), lambda i: (0, 0)),
            pl.BlockSpec((1, _D_H), lambda i: (0, 0)),
            pl.BlockSpec((1, _D_H), lambda i: (0, 0)),
            pl.BlockSpec((1, 1), lambda i: (0, 0)),
        ],
        out_specs=pl.BlockSpec((_TB,), lambda i: (i,)),
        out_shape=jax.ShapeDtypeStruct((_BATCH,), jnp.float32),
        compiler_params=pltpu.CompilerParams(
            dimension_semantics=("arbitrary",),
        ),
    )(alpha, beta, W1, b1r, W2, b2r, w3r, b3r)
    return out
